# baseline (speedup 1.0000x reference)
import functools

import jax
import jax.numpy as jnp
from jax import lax
from jax.experimental import pallas as pl
from jax.experimental.pallas import tpu as pltpu
from jax.experimental.pallas import tpu_sc as plsc

N = 64
NBINS = 256
W = 512
RPS = 3 * 512
RPC = 64
CPS = RPS // RPC
SCALE = 200.0 / float(512 * 512)


def _sc_hist_body(x_hbm, out_hbm, buf0, buf1, hist, outbuf, sem0, sem1):
    nc = lax.axis_size("c")
    nw = nc * lax.axis_size("s")
    wid = lax.axis_index("s") * nc + lax.axis_index("c")
    spw = N // nw
    offs = lax.iota(jnp.int32, 16) * NBINS
    ones = jnp.ones((16,), jnp.float32)
    zeros16 = jnp.zeros((16,), jnp.float32)
    bufs = [buf0, buf1]
    sems = [sem0, sem1]

    for sl in range(spw):
        samp = wid * spw + sl
        rbase = samp * RPS

        @plsc.parallel_loop(0, 16 * NBINS // 16)
        def _zero(j):
            hist[pl.ds(j * 16, 16)] = zeros16

        descs = [None, None]
        descs[0] = pltpu.async_copy(
            x_hbm.at[pl.ds(rbase, RPC)], bufs[0], sems[0])
        for t in range(CPS):
            b = t % 2
            if t + 1 < CPS:
                descs[1 - b] = pltpu.async_copy(
                    x_hbm.at[pl.ds(rbase + (t + 1) * RPC, RPC)],
                    bufs[1 - b], sems[1 - b])
            descs[b].wait()
            buf = bufs[b]

            @plsc.parallel_loop(0, RPC * W // 16, unroll=8)
            def _bin(i):
                r = lax.shift_right_logical(i, 5)
                c = lax.shift_left(jnp.bitwise_and(i, 31), 4)
                v = buf[r, pl.ds(c, 16)]
                plsc.addupdate_scatter(
                    hist, [(v * 256.0).astype(jnp.int32) + offs], ones)

        @plsc.parallel_loop(0, NBINS // 16)
        def _red(j):
            acc = zeros16
            for l in range(16):
                acc = acc + hist[pl.ds(l * NBINS + j * 16, 16)]
            outbuf[pl.ds(j * 16, 16)] = acc * SCALE

        pltpu.sync_copy(outbuf, out_hbm.at[pl.ds(samp * NBINS, NBINS)])


def _sc_histograms(x2d):
    mesh = plsc.VectorSubcoreMesh(core_axis_name="c", subcore_axis_name="s")
    call = pl.kernel(
        _sc_hist_body,
        out_type=jax.ShapeDtypeStruct((N * NBINS,), jnp.float32),
        mesh=mesh,
        compiler_params=pltpu.CompilerParams(needs_layout_passes=False),
        scratch_types=[
            pltpu.VMEM((RPC, W), jnp.float32),
            pltpu.VMEM((RPC, W), jnp.float32),
            pltpu.VMEM((16 * NBINS,), jnp.float32),
            pltpu.VMEM((NBINS,), jnp.float32),
            pltpu.SemaphoreType.DMA,
            pltpu.SemaphoreType.DMA,
        ],
    )
    return call(x2d)


def _tc_head_body(h_ref, b1_ref, c1b_ref, b2_ref, c2b_ref,
                  fc1_ref, fc1b_ref, fc2_ref, fc2b_ref, out_ref):
    hp = lax.Precision.HIGHEST
    h = h_ref[:]
    h = jnp.maximum(
        lax.dot_general(h, b1_ref[:], (((1,), (0,)), ((), ())), precision=hp)
        + c1b_ref[:], 0.0)
    h = jnp.maximum(
        lax.dot_general(h, b2_ref[:], (((1,), (0,)), ((), ())), precision=hp)
        + c2b_ref[:], 0.0)
    h = jnp.maximum(
        lax.dot_general(h, fc1_ref[:], (((1,), (1,)), ((), ())), precision=hp)
        + fc1b_ref[:], 0.0)
    out_ref[:] = jnp.maximum(
        lax.dot_general(h, fc2_ref[:], (((1,), (1,)), ((), ())), precision=hp)
        + fc2b_ref[:], 0.0)


def _band_matrix(w7):
    r = jnp.arange(NBINS)
    diff = r[:, None] - r[None, :]
    return jnp.where(jnp.abs(diff) <= 3, w7[jnp.clip(diff + 3, 0, 6)], 0.0)


def kernel(x, conv1_w, conv1_b, conv2_w, conv2_b, fc1_w, fc1_b, fc2_w, fc2_b):
    x2d = x.reshape(N * RPS, W)
    hist = _sc_histograms(x2d).reshape(N, NBINS)

    b1 = _band_matrix(conv1_w.reshape(7))
    b2 = _band_matrix(conv2_w.reshape(7))
    out = pl.pallas_call(
        _tc_head_body,
        out_shape=jax.ShapeDtypeStruct((N, NBINS), jnp.float32),
    )(hist, b1, conv1_b.reshape(1, 1), b2, conv2_b.reshape(1, 1),
      fc1_w, fc1_b.reshape(1, NBINS), fc2_w, fc2_b.reshape(1, NBINS))
    return out


# speedup vs baseline: 1.0231x; 1.0231x over previous
import functools

import jax
import jax.numpy as jnp
from jax import lax
from jax.experimental import pallas as pl
from jax.experimental.pallas import tpu as pltpu
from jax.experimental.pallas import tpu_sc as plsc

N = 64
NBINS = 256
W = 512
RPS = 3 * 512
RPC = 64
CPS = RPS // RPC
SCALE = 200.0 / float(512 * 512)


def _sc_hist_body(x_hbm, out_hbm, buf0, buf1, hist, outbuf, sem0, sem1):
    nc = lax.axis_size("c")
    nw = nc * lax.axis_size("s")
    wid = lax.axis_index("s") * nc + lax.axis_index("c")
    spw = N // nw
    offs = lax.iota(jnp.int32, 16) * NBINS
    ones = jnp.ones((16,), jnp.float32)
    zeros16 = jnp.zeros((16,), jnp.float32)
    bufs = [buf0, buf1]
    sems = [sem0, sem1]

    for sl in range(spw):
        samp = wid * spw + sl
        rbase = samp * RPS

        @plsc.parallel_loop(0, 16 * NBINS // 16)
        def _zero(j):
            hist[pl.ds(j * 16, 16)] = zeros16

        descs = [None, None]
        descs[0] = pltpu.async_copy(
            x_hbm.at[pl.ds(rbase, RPC)], bufs[0], sems[0])
        for t in range(CPS):
            b = t % 2
            if t + 1 < CPS:
                descs[1 - b] = pltpu.async_copy(
                    x_hbm.at[pl.ds(rbase + (t + 1) * RPC, RPC)],
                    bufs[1 - b], sems[1 - b])
            descs[b].wait()
            buf = bufs[b]

            @plsc.parallel_loop(0, RPC * W // 16, unroll=8)
            def _bin(i):
                r = lax.shift_right_logical(i, 5)
                c = lax.shift_left(jnp.bitwise_and(i, 31), 4)
                v = buf[r, pl.ds(c, 16)]
                bb = jnp.minimum(jnp.maximum(v * 256.0, 0.0), 255.0)
                plsc.addupdate_scatter(
                    hist, [bb.astype(jnp.int32) + offs], ones)

        @plsc.parallel_loop(0, NBINS // 16)
        def _red(j):
            acc = zeros16
            for l in range(16):
                acc = acc + hist[pl.ds(l * NBINS + j * 16, 16)]
            outbuf[pl.ds(j * 16, 16)] = acc * SCALE

        pltpu.sync_copy(outbuf, out_hbm.at[pl.ds(samp * NBINS, NBINS)])


def _sc_histograms(x2d):
    mesh = plsc.VectorSubcoreMesh(core_axis_name="c", subcore_axis_name="s")
    call = pl.kernel(
        _sc_hist_body,
        out_type=jax.ShapeDtypeStruct((N * NBINS,), jnp.float32),
        mesh=mesh,
        compiler_params=pltpu.CompilerParams(needs_layout_passes=False),
        scratch_types=[
            pltpu.VMEM((RPC, W), jnp.float32),
            pltpu.VMEM((RPC, W), jnp.float32),
            pltpu.VMEM((16 * NBINS,), jnp.float32),
            pltpu.VMEM((NBINS,), jnp.float32),
            pltpu.SemaphoreType.DMA,
            pltpu.SemaphoreType.DMA,
        ],
    )
    return call(x2d)


def _tc_head_body(h_ref, b1_ref, c1b_ref, b2_ref, c2b_ref,
                  fc1_ref, fc1b_ref, fc2_ref, fc2b_ref, out_ref):
    hp = None
    h = h_ref[:]
    h = jnp.maximum(
        lax.dot_general(h, b1_ref[:], (((1,), (0,)), ((), ())), precision=hp)
        + c1b_ref[:], 0.0)
    h = jnp.maximum(
        lax.dot_general(h, b2_ref[:], (((1,), (0,)), ((), ())), precision=hp)
        + c2b_ref[:], 0.0)
    h = jnp.maximum(
        lax.dot_general(h, fc1_ref[:], (((1,), (1,)), ((), ())), precision=hp)
        + fc1b_ref[:], 0.0)
    out_ref[:] = jnp.maximum(
        lax.dot_general(h, fc2_ref[:], (((1,), (1,)), ((), ())), precision=hp)
        + fc2b_ref[:], 0.0)


def _band_matrix(w7):
    r = jnp.arange(NBINS)
    diff = r[:, None] - r[None, :]
    return jnp.where(jnp.abs(diff) <= 3, w7[jnp.clip(diff + 3, 0, 6)], 0.0)


def kernel(x, conv1_w, conv1_b, conv2_w, conv2_b, fc1_w, fc1_b, fc2_w, fc2_b):
    x2d = x.reshape(N * RPS, W)
    hist = _sc_histograms(x2d).reshape(N, NBINS)

    b1 = _band_matrix(conv1_w.reshape(7))
    b2 = _band_matrix(conv2_w.reshape(7))
    out = pl.pallas_call(
        _tc_head_body,
        out_shape=jax.ShapeDtypeStruct((N, NBINS), jnp.float32),
    )(hist, b1, conv1_b.reshape(1, 1), b2, conv2_b.reshape(1, 1),
      fc1_w, fc1_b.reshape(1, NBINS), fc2_w, fc2_b.reshape(1, NBINS))
    return out
